# Initial kernel scaffold; baseline (speedup 1.0000x reference)
import jax
import jax.numpy as jnp
from jax.experimental import pallas as pl


def kernel(x, edge_index, edge_weight, params):
    raise NotImplementedError("write your pallas kernel here")


# speedup vs baseline: 1.0000x; 1.0000x over previous
import jax
import jax.numpy as jnp
from jax.experimental import pallas as pl

SIGMA = 0.5
V = 1.0


def _bn(x, g, b, eps=1e-5):
    mu = jnp.mean(x, axis=0, keepdims=True)
    var = jnp.mean((x - mu) ** 2, axis=0, keepdims=True)
    return g * (x - mu) / jnp.sqrt(var + eps) + b


def _spmm(edge_index, edge_weight, support):
    row, col = edge_index[0], edge_index[1]
    msgs = support[col] * edge_weight[:, None]
    return jax.ops.segment_sum(msgs, row, num_segments=support.shape[0])


def _gnn(x, edge_index, edge_weight, W, active=True):
    out = _spmm(edge_index, edge_weight, x @ W)
    return jax.nn.relu(out) if active else out


def _ae(p, x):
    h1 = jax.nn.relu(_bn(x @ p["enc_1_W"] + p["enc_1_b"], p["BN1_g"], p["BN1_b"]))
    h2 = jax.nn.relu(_bn(h1 @ p["enc_2_W"] + p["enc_2_b"], p["BN2_g"], p["BN2_b"]))
    h3 = jax.nn.relu(_bn(h2 @ p["enc_3_W"] + p["enc_3_b"], p["BN3_g"], p["BN3_b"]))
    z1 = _bn(h3 @ p["z1_W"] + p["z1_b"], p["BN4_g"], p["BN4_b"])
    z2 = _bn(z1 @ p["z2_W"] + p["z2_b"], p["BN5_g"], p["BN5_b"])
    z3 = _bn(z2 @ p["z3_W"] + p["z3_b"], p["BN6_g"], p["BN6_b"])
    d1 = jax.nn.relu(_bn(z3 @ p["dec_1_W"] + p["dec_1_b"], p["BN7_g"], p["BN7_b"]))
    d2 = jax.nn.relu(_bn(d1 @ p["dec_2_W"] + p["dec_2_b"], p["BN8_g"], p["BN8_b"]))
    d3 = jax.nn.relu(_bn(d2 @ p["dec_3_W"] + p["dec_3_b"], p["BN9_g"], p["BN9_b"]))
    x_bar = d3 @ p["x_bar_W"] + p["x_bar_b"]
    return x_bar, h1, h2, h3, z3, z2, z1, d3


def _copy_kernel(x_ref, o_ref):
    o_ref[...] = x_ref[...]


def _pl_identity(x):
    return pl.pallas_call(
        _copy_kernel,
        out_shape=jax.ShapeDtypeStruct(x.shape, x.dtype),
    )(x)


def kernel(x, edge_index, edge_weight, params):
    p = params
    x_bar, tra1, tra2, tra3, z3, z2, z1, dec_h3 = _ae(p, x)
    s = SIGMA
    h = _gnn(x, edge_index, edge_weight, p["gnn_1_W"])
    h = _gnn((1 - s) * h + s * tra1, edge_index, edge_weight, p["gnn_2_W"])
    h = _gnn((1 - s) * h + s * tra2, edge_index, edge_weight, p["gnn_3_W"])
    h = _gnn((1 - s) * h + s * tra3, edge_index, edge_weight, p["gnn_4_W"])
    h = _gnn((1 - s) * h + s * z1, edge_index, edge_weight, p["gnn_5_W"])
    h = _gnn((1 - s) * h + s * z2, edge_index, edge_weight, p["gnn_6_W"])
    h = _gnn((1 - s) * h + s * z3, edge_index, edge_weight, p["gnn_7_W"], active=False)
    predict = jax.nn.softmax(h, axis=1)
    mean = jnp.clip(jnp.exp(dec_h3 @ p["dmean_W"] + p["dmean_b"]), 1e-5, 1e6)
    disp = jnp.clip(jax.nn.softplus(dec_h3 @ p["ddisp_W"] + p["ddisp_b"]), 1e-4, 1e4)
    pi = jax.nn.sigmoid(dec_h3 @ p["dpi_W"] + p["dpi_b"])
    q = 1.0 / (1.0 + jnp.sum((z3[:, None, :] - p["cluster"]) ** 2, axis=2) / V)
    q = q ** ((V + 1.0) / 2.0)
    q = q / jnp.sum(q, axis=1, keepdims=True)
    x_bar = _pl_identity(x_bar)
    return (x_bar, q, predict, z3, mean, disp, pi)


# speedup vs baseline: 4.2629x; 4.2629x over previous
import dataclasses
import functools

import jax
import jax.numpy as jnp
from jax import lax
from jax.experimental import pallas as pl
from jax.experimental.pallas import tpu as pltpu
from jax.experimental.pallas import tpu_sc as plsc

SIGMA = 0.5
V = 1.0
N_NODES = 10000
N_EDGES = 160000
NTILES = 32
EDGES_PER_TILE = N_EDGES // NTILES
EB = 200
NB = EDGES_PER_TILE // EB
ROWS_PER_TILE = N_NODES // 16


def _linbn_body(x_ref, w_ref, b_ref, g_ref, bb_ref, o_ref, *, relu):
    y = jnp.dot(x_ref[...], w_ref[...], preferred_element_type=jnp.float32)
    y = y + b_ref[...]
    mu = jnp.mean(y, axis=0, keepdims=True)
    var = jnp.mean((y - mu) ** 2, axis=0, keepdims=True)
    y = g_ref[...] * (y - mu) / jnp.sqrt(var + 1e-5) + bb_ref[...]
    if relu:
        y = jnp.maximum(y, 0.0)
    o_ref[...] = y


def _lin_bn(x, w, b, g, bb, relu):
    n, k = x.shape
    fo = w.shape[1]
    bw = 128 if fo % 128 == 0 else fo
    grid = (fo // bw,)
    return pl.pallas_call(
        functools.partial(_linbn_body, relu=relu),
        grid=grid,
        in_specs=[
            pl.BlockSpec((n, k), lambda c: (0, 0)),
            pl.BlockSpec((k, bw), lambda c: (0, c)),
            pl.BlockSpec((bw,), lambda c: (c,)),
            pl.BlockSpec((bw,), lambda c: (c,)),
            pl.BlockSpec((bw,), lambda c: (c,)),
        ],
        out_specs=pl.BlockSpec((n, bw), lambda c: (0, c)),
        out_shape=jax.ShapeDtypeStruct((n, fo), jnp.float32),
    )(x, w, b, g, bb)


def _mm_chunk_body(h_ref, w_ref, o_ref):
    o_ref[0] = jnp.dot(h_ref[...], w_ref[...], preferred_element_type=jnp.float32)


def _matmul_chunked(h, w, fc):
    n, k = h.shape
    fo = w.shape[1]
    nch = fo // fc
    return pl.pallas_call(
        _mm_chunk_body,
        grid=(nch,),
        in_specs=[
            pl.BlockSpec((n, k), lambda c: (0, 0)),
            pl.BlockSpec((k, fc), lambda c: (0, c)),
        ],
        out_specs=pl.BlockSpec((1, n, fc), lambda c: (c, 0, 0)),
        out_shape=jax.ShapeDtypeStruct((nch, n, fc), jnp.float32),
    )(h, w)


def _postmm_body(p_ref, w_ref, t_ref, o_ref, *, nchp):
    h = p_ref[0, 0] + p_ref[1, 0]
    if nchp > 1:
        h = jnp.concatenate(
            [h] + [p_ref[0, c] + p_ref[1, c] for c in range(1, nchp)], axis=1)
    y = jnp.maximum(jnp.dot(h, w_ref[...], preferred_element_type=jnp.float32),
                    0.0)
    o_ref[...] = (1.0 - SIGMA) * y + SIGMA * t_ref[...]


def _postmm_mix(part, w, tra):
    _, nchp, n, fcp = part.shape
    fo = w.shape[1]
    bw = 128 if fo % 128 == 0 else fo
    return pl.pallas_call(
        functools.partial(_postmm_body, nchp=nchp),
        grid=(fo // bw,),
        in_specs=[
            pl.BlockSpec((2, nchp, n, fcp), lambda c: (0, 0, 0, 0)),
            pl.BlockSpec((nchp * fcp, bw), lambda c: (0, c)),
            pl.BlockSpec((n, bw), lambda c: (0, c)),
        ],
        out_specs=pl.BlockSpec((n, bw), lambda c: (0, c)),
        out_shape=jax.ShapeDtypeStruct((n, fo), jnp.float32),
    )(part, w, tra)


def _mix_body(p0_ref, p1_ref, t_ref, o_ref, *, kk):
    h = p0_ref[0, 0] + p1_ref[0, 0]
    if kk > 1:
        h = jnp.concatenate(
            [h] + [p0_ref[0, c] + p1_ref[0, c] for c in range(1, kk)], axis=1)
    h = jnp.maximum(h, 0.0)
    o_ref[...] = (1.0 - SIGMA) * h + SIGMA * t_ref[...]


def _mix(part, tra):
    _, nch, n, fc = part.shape
    fo = nch * fc
    bw = 128 if fo % 128 == 0 else fo
    kk = bw // fc
    return pl.pallas_call(
        functools.partial(_mix_body, kk=kk),
        grid=(fo // bw,),
        in_specs=[
            pl.BlockSpec((1, kk, n, fc), lambda c: (0, c, 0, 0)),
            pl.BlockSpec((1, kk, n, fc), lambda c: (1, c, 0, 0)),
            pl.BlockSpec((n, bw), lambda c: (0, c)),
        ],
        out_specs=pl.BlockSpec((n, bw), lambda c: (0, c)),
        out_shape=jax.ShapeDtypeStruct((n, fo), jnp.float32),
    )(part[:, :, :, :], part, tra)


def _softmax_body(p0_ref, p1_ref, o_ref):
    h = p0_ref[...] + p1_ref[...]
    m = jnp.max(h, axis=1, keepdims=True)
    e = jnp.exp(h - m)
    o_ref[...] = e / jnp.sum(e, axis=1, keepdims=True)


def _softmax_of_part(part):
    _, nch, n, fc = part.shape
    return pl.pallas_call(
        _softmax_body,
        out_shape=jax.ShapeDtypeStruct((n, fc), jnp.float32),
    )(part[0, 0], part[1, 0])


def _heads_body(d_ref, wx_ref, bx_ref, wm_ref, bm_ref, wd_ref, bd_ref,
                wp_ref, bp_ref, xb_ref, mean_ref, disp_ref, pi_ref):
    d = d_ref[...]
    xb_ref[...] = jnp.dot(d, wx_ref[...], preferred_element_type=jnp.float32) + bx_ref[...]
    ym = jnp.dot(d, wm_ref[...], preferred_element_type=jnp.float32) + bm_ref[...]
    mean_ref[...] = jnp.clip(jnp.exp(ym), 1e-5, 1e6)
    yd = jnp.dot(d, wd_ref[...], preferred_element_type=jnp.float32) + bd_ref[...]
    disp_ref[...] = jnp.clip(jnp.log1p(jnp.exp(-jnp.abs(yd))) + jnp.maximum(yd, 0.0),
                             1e-4, 1e4)
    yp = jnp.dot(d, wp_ref[...], preferred_element_type=jnp.float32) + bp_ref[...]
    pi_ref[...] = 1.0 / (1.0 + jnp.exp(-yp))


def _heads(d3, p):
    n = d3.shape[0]
    fo = p["x_bar_W"].shape[1]
    outs = [jax.ShapeDtypeStruct((n, fo), jnp.float32)] * 4
    return pl.pallas_call(
        _heads_body,
        out_shape=outs,
    )(d3, p["x_bar_W"], p["x_bar_b"], p["dmean_W"], p["dmean_b"],
      p["ddisp_W"], p["ddisp_b"], p["dpi_W"], p["dpi_b"])


def _q_body(z_ref, ct_ref, o_ref):
    z = z_ref[...]
    ct = ct_ref[...]
    zz = jnp.sum(z * z, axis=1, keepdims=True)
    cc = jnp.sum(ct * ct, axis=0, keepdims=True)
    cross = jnp.dot(z, ct, preferred_element_type=jnp.float32)
    q = 1.0 / (1.0 + (zz + cc - 2.0 * cross) / V)
    o_ref[...] = q / jnp.sum(q, axis=1, keepdims=True)


def _q_kernel(z3, cluster_t):
    n = z3.shape[0]
    nc = cluster_t.shape[1]
    return pl.pallas_call(
        _q_body,
        out_shape=jax.ShapeDtypeStruct((n, nc), jnp.float32),
    )(z3, cluster_t)


def _sc_spmm(tbl_chunks, col3, row3, w2, zeros, width):
    nch = len(tbl_chunks)
    n = 16 * ROWS_PER_TILE
    ngroups = width // 16
    mesh = plsc.VectorSubcoreMesh(core_axis_name="c", subcore_axis_name="s")
    cp = pltpu.CompilerParams()
    fields = pltpu.CompilerParams.__dataclass_fields__
    if "needs_layout_passes" in fields:
        cp = dataclasses.replace(cp, needs_layout_passes=False)

    @functools.partial(
        pl.kernel,
        compiler_params=cp,
        out_type=jax.ShapeDtypeStruct((2, nch, 16, ROWS_PER_TILE, 128),
                                      jnp.float32),
        mesh=mesh,
        scratch_types=[
            pltpu.VMEM((NB, 1, EB), jnp.int32),
            pltpu.VMEM((NB, 1, EB), jnp.int32),
            pltpu.VMEM((NB, EB), jnp.float32),
            pltpu.VMEM((EB, 128), jnp.float32),
            pltpu.VMEM_SHARED((n, 128), jnp.float32),
            pltpu.SemaphoreType.DMA,
        ],
    )
    def k(*refs):
        tbls = refs[:nch]
        col_h, row_h, w_h, z_h, out_h = refs[nch:nch + 5]
        col_v, row_v, w_v, rows_v, acc, sem = refs[nch + 5:]
        core = lax.axis_index("c")
        sub = lax.axis_index("s")
        wid = sub * 2 + core
        rslice = pl.ds(sub * ROWS_PER_TILE, ROWS_PER_TILE)
        pltpu.sync_copy(col_h.at[wid], col_v)
        pltpu.sync_copy(row_h.at[wid], row_v)
        pltpu.sync_copy(w_h.at[wid], w_v)
        for ch in range(nch):
            pltpu.sync_copy(z_h.at[sub], acc.at[rslice])
            plsc.subcore_barrier()

            @pl.loop(0, NB)
            def _batch(b):
                pltpu.async_copy(tbls[ch].at[col_v.at[b].at[0]], rows_v,
                                 sem).wait()

                @pl.loop(0, EB)
                def _edge(e):
                    wv = plsc.load_gather(
                        w_v, [jnp.full((16,), b, jnp.int32),
                              jnp.full((16,), e, jnp.int32)])
                    for f in range(ngroups):
                        sl = pl.ds(f * 16, 16)
                        rows_v[e, sl] = rows_v[e, sl] * wv

                pltpu.sync_copy(rows_v, acc.at[row_v.at[b].at[0]], add=True)

            plsc.subcore_barrier()
            pltpu.sync_copy(acc.at[rslice], out_h.at[core, ch, sub])
            plsc.subcore_barrier()

    part = k(*tbl_chunks, col3, row3, w2, zeros)
    return part.reshape(2, nch, n, 128)


def kernel(x, edge_index, edge_weight, params):
    p = params
    n = x.shape[0]

    col3 = edge_index[1].reshape(NTILES, NB, 1, EB)
    row3 = edge_index[0].reshape(NTILES, NB, 1, EB)
    w2 = edge_weight.reshape(NTILES, NB, EB)

    h1 = _lin_bn(x, p["enc_1_W"], p["enc_1_b"], p["BN1_g"], p["BN1_b"], True)
    h2 = _lin_bn(h1, p["enc_2_W"], p["enc_2_b"], p["BN2_g"], p["BN2_b"], True)
    h3 = _lin_bn(h2, p["enc_3_W"], p["enc_3_b"], p["BN3_g"], p["BN3_b"], True)
    z1 = _lin_bn(h3, p["z1_W"], p["z1_b"], p["BN4_g"], p["BN4_b"], False)
    z2 = _lin_bn(z1, p["z2_W"], p["z2_b"], p["BN5_g"], p["BN5_b"], False)
    z3 = _lin_bn(z2, p["z3_W"], p["z3_b"], p["BN6_g"], p["BN6_b"], False)
    d1 = _lin_bn(z3, p["dec_1_W"], p["dec_1_b"], p["BN7_g"], p["BN7_b"], True)
    d2 = _lin_bn(d1, p["dec_2_W"], p["dec_2_b"], p["BN8_g"], p["BN8_b"], True)
    d3 = _lin_bn(d2, p["dec_3_W"], p["dec_3_b"], p["BN9_g"], p["BN9_b"], True)
    x_bar, mean, disp, pi = _heads(d3, p)

    tras = [h1, h2, h3, z1, z2, z3]
    zeros = jnp.zeros((16, ROWS_PER_TILE, 128), jnp.float32)

    part = _sc_spmm([x], col3, row3, w2, zeros, 128)
    hmix = _postmm_mix(part, p["gnn_1_W"], tras[0])
    for i in range(2, 8):
        gw = p["gnn_%d_W" % i]
        fo = gw.shape[1]
        bw = 128 if fo % 128 == 0 else fo
        tm = _matmul_chunked(hmix, gw, bw)
        if bw < 128:
            chunks = [jnp.pad(tm[0], ((0, 0), (0, 128 - bw)))]
        else:
            chunks = [tm[c] for c in range(fo // 128)]
        part = _sc_spmm(chunks, col3, row3, w2, zeros, bw)
        part = part[:, :, :, :bw]
        if i < 7:
            hmix = _mix(part, tras[i - 1])
    predict = _softmax_of_part(part)

    q = _q_kernel(z3, p["cluster"].T)

    return (x_bar, q, predict, z3, mean, disp, pi)


# speedup vs baseline: 5.8042x; 1.3615x over previous
import dataclasses
import functools

import jax
import jax.numpy as jnp
from jax import lax
from jax.experimental import pallas as pl
from jax.experimental.pallas import tpu as pltpu
from jax.experimental.pallas import tpu_sc as plsc

SIGMA = 0.5
V = 1.0
N_NODES = 10000
N_EDGES = 160000
NTILES = 32
EDGES_PER_TILE = N_EDGES // NTILES
EB = 250
NB = EDGES_PER_TILE // EB
ROWS_PER_TILE = N_NODES // 16


def _linbn_body(x_ref, w_ref, b_ref, g_ref, bb_ref, o_ref, *, relu):
    y = jnp.dot(x_ref[...], w_ref[...], preferred_element_type=jnp.float32)
    y = y + b_ref[...]
    mu = jnp.mean(y, axis=0, keepdims=True)
    var = jnp.mean((y - mu) ** 2, axis=0, keepdims=True)
    y = g_ref[...] * (y - mu) / jnp.sqrt(var + 1e-5) + bb_ref[...]
    if relu:
        y = jnp.maximum(y, 0.0)
    o_ref[...] = y


def _lin_bn(x, w, b, g, bb, relu):
    n, k = x.shape
    fo = w.shape[1]
    bw = 128 if fo % 128 == 0 else fo
    grid = (fo // bw,)
    return pl.pallas_call(
        functools.partial(_linbn_body, relu=relu),
        grid=grid,
        in_specs=[
            pl.BlockSpec((n, k), lambda c: (0, 0)),
            pl.BlockSpec((k, bw), lambda c: (0, c)),
            pl.BlockSpec((bw,), lambda c: (c,)),
            pl.BlockSpec((bw,), lambda c: (c,)),
            pl.BlockSpec((bw,), lambda c: (c,)),
        ],
        out_specs=pl.BlockSpec((n, bw), lambda c: (0, c)),
        out_shape=jax.ShapeDtypeStruct((n, fo), jnp.float32),
    )(x, w, b, g, bb)


def _mm_chunk_body(h_ref, w_ref, o_ref):
    o_ref[0] = jnp.dot(h_ref[...], w_ref[...], preferred_element_type=jnp.float32)


def _matmul_chunked(h, w, fc):
    n, k = h.shape
    fo = w.shape[1]
    nch = fo // fc
    return pl.pallas_call(
        _mm_chunk_body,
        grid=(nch,),
        in_specs=[
            pl.BlockSpec((n, k), lambda c: (0, 0)),
            pl.BlockSpec((k, fc), lambda c: (0, c)),
        ],
        out_specs=pl.BlockSpec((1, n, fc), lambda c: (c, 0, 0)),
        out_shape=jax.ShapeDtypeStruct((nch, n, fc), jnp.float32),
    )(h, w)


def _postmm_body(p_ref, w_ref, t_ref, o_ref, *, nchp):
    h = p_ref[0, 0] + p_ref[1, 0]
    if nchp > 1:
        h = jnp.concatenate(
            [h] + [p_ref[0, c] + p_ref[1, c] for c in range(1, nchp)], axis=1)
    y = jnp.maximum(jnp.dot(h, w_ref[...], preferred_element_type=jnp.float32),
                    0.0)
    o_ref[...] = (1.0 - SIGMA) * y + SIGMA * t_ref[...]


def _postmm_mix(part, w, tra):
    _, nchp, n, fcp = part.shape
    fo = w.shape[1]
    bw = 128 if fo % 128 == 0 else fo
    return pl.pallas_call(
        functools.partial(_postmm_body, nchp=nchp),
        grid=(fo // bw,),
        in_specs=[
            pl.BlockSpec((2, nchp, n, fcp), lambda c: (0, 0, 0, 0)),
            pl.BlockSpec((nchp * fcp, bw), lambda c: (0, c)),
            pl.BlockSpec((n, bw), lambda c: (0, c)),
        ],
        out_specs=pl.BlockSpec((n, bw), lambda c: (0, c)),
        out_shape=jax.ShapeDtypeStruct((n, fo), jnp.float32),
    )(part, w, tra)


def _mix_body(p0_ref, p1_ref, t_ref, o_ref, *, kk):
    h = p0_ref[0, 0] + p1_ref[0, 0]
    if kk > 1:
        h = jnp.concatenate(
            [h] + [p0_ref[0, c] + p1_ref[0, c] for c in range(1, kk)], axis=1)
    h = jnp.maximum(h, 0.0)
    o_ref[...] = (1.0 - SIGMA) * h + SIGMA * t_ref[...]


def _mix(part, tra):
    _, nch, n, fc = part.shape
    fo = nch * fc
    bw = 128 if fo % 128 == 0 else fo
    kk = bw // fc
    return pl.pallas_call(
        functools.partial(_mix_body, kk=kk),
        grid=(fo // bw,),
        in_specs=[
            pl.BlockSpec((1, kk, n, fc), lambda c: (0, c, 0, 0)),
            pl.BlockSpec((1, kk, n, fc), lambda c: (1, c, 0, 0)),
            pl.BlockSpec((n, bw), lambda c: (0, c)),
        ],
        out_specs=pl.BlockSpec((n, bw), lambda c: (0, c)),
        out_shape=jax.ShapeDtypeStruct((n, fo), jnp.float32),
    )(part[:, :, :, :], part, tra)


def _softmax_body(p0_ref, p1_ref, o_ref):
    h = p0_ref[...] + p1_ref[...]
    m = jnp.max(h, axis=1, keepdims=True)
    e = jnp.exp(h - m)
    o_ref[...] = e / jnp.sum(e, axis=1, keepdims=True)


def _softmax_of_part(part):
    _, nch, n, fc = part.shape
    return pl.pallas_call(
        _softmax_body,
        out_shape=jax.ShapeDtypeStruct((n, fc), jnp.float32),
    )(part[0, 0], part[1, 0])


def _heads_body(d_ref, wx_ref, bx_ref, wm_ref, bm_ref, wd_ref, bd_ref,
                wp_ref, bp_ref, xb_ref, mean_ref, disp_ref, pi_ref):
    d = d_ref[...]
    xb_ref[...] = jnp.dot(d, wx_ref[...], preferred_element_type=jnp.float32) + bx_ref[...]
    ym = jnp.dot(d, wm_ref[...], preferred_element_type=jnp.float32) + bm_ref[...]
    mean_ref[...] = jnp.clip(jnp.exp(ym), 1e-5, 1e6)
    yd = jnp.dot(d, wd_ref[...], preferred_element_type=jnp.float32) + bd_ref[...]
    disp_ref[...] = jnp.clip(jnp.log1p(jnp.exp(-jnp.abs(yd))) + jnp.maximum(yd, 0.0),
                             1e-4, 1e4)
    yp = jnp.dot(d, wp_ref[...], preferred_element_type=jnp.float32) + bp_ref[...]
    pi_ref[...] = 1.0 / (1.0 + jnp.exp(-yp))


def _heads(d3, p):
    n = d3.shape[0]
    fo = p["x_bar_W"].shape[1]
    outs = [jax.ShapeDtypeStruct((n, fo), jnp.float32)] * 4
    return pl.pallas_call(
        _heads_body,
        out_shape=outs,
    )(d3, p["x_bar_W"], p["x_bar_b"], p["dmean_W"], p["dmean_b"],
      p["ddisp_W"], p["ddisp_b"], p["dpi_W"], p["dpi_b"])


def _q_body(z_ref, ct_ref, o_ref):
    z = z_ref[...]
    ct = ct_ref[...]
    zz = jnp.sum(z * z, axis=1, keepdims=True)
    cc = jnp.sum(ct * ct, axis=0, keepdims=True)
    cross = jnp.dot(z, ct, preferred_element_type=jnp.float32)
    q = 1.0 / (1.0 + (zz + cc - 2.0 * cross) / V)
    o_ref[...] = q / jnp.sum(q, axis=1, keepdims=True)


def _q_kernel(z3, cluster_t):
    n = z3.shape[0]
    nc = cluster_t.shape[1]
    return pl.pallas_call(
        _q_body,
        out_shape=jax.ShapeDtypeStruct((n, nc), jnp.float32),
    )(z3, cluster_t)


def _sc_spmm(tbl_chunks, col3, row3, w2, zeros, width):
    nch = len(tbl_chunks)
    n = 16 * ROWS_PER_TILE
    ngroups = width // 16
    mesh = plsc.VectorSubcoreMesh(core_axis_name="c", subcore_axis_name="s")
    cp = pltpu.CompilerParams()
    fields = pltpu.CompilerParams.__dataclass_fields__
    if "needs_layout_passes" in fields:
        cp = dataclasses.replace(cp, needs_layout_passes=False)

    @functools.partial(
        pl.kernel,
        compiler_params=cp,
        out_type=jax.ShapeDtypeStruct((2, nch, 16, ROWS_PER_TILE, 128),
                                      jnp.float32),
        mesh=mesh,
        scratch_types=[
            pltpu.VMEM((NB, 1, EB), jnp.int32),
            pltpu.VMEM((NB, 1, EB), jnp.int32),
            pltpu.VMEM((NB, EB), jnp.float32),
            pltpu.VMEM((EB, 128), jnp.float32),
            pltpu.VMEM_SHARED((n, 128), jnp.float32),
            pltpu.SemaphoreType.DMA,
        ],
    )
    def k(*refs):
        tbls = refs[:nch]
        col_h, row_h, w_h, z_h, out_h = refs[nch:nch + 5]
        col_v, row_v, w_v, rv0, acc, sg0 = refs[nch + 5:]
        core = lax.axis_index("c")
        sub = lax.axis_index("s")
        wid = sub * 2 + core
        rslice = pl.ds(sub * ROWS_PER_TILE, ROWS_PER_TILE)
        pltpu.sync_copy(col_h.at[wid], col_v)
        pltpu.sync_copy(row_h.at[wid], row_v)
        pltpu.sync_copy(w_h.at[wid], w_v)

        for ch in range(nch):
            tb = tbls[ch]

            def scale(b, rv):
                @functools.partial(plsc.parallel_loop, 0, EB, unroll=5)
                def _edge(e):
                    wv = plsc.load_gather(
                        w_v, [jnp.full((16,), b, jnp.int32),
                              jnp.full((16,), e, jnp.int32)])
                    for f in range(ngroups):
                        sl = pl.ds(f * 16, 16)
                        rv[e, sl] = rv[e, sl] * wv

            pltpu.sync_copy(z_h, acc.at[rslice])
            plsc.subcore_barrier()

            @pl.loop(0, NB)
            def _batch(b):
                pltpu.async_copy(tb.at[col_v.at[b].at[0]], rv0, sg0).wait()
                scale(b, rv0)
                pltpu.sync_copy(rv0, acc.at[row_v.at[b].at[0]], add=True)

            plsc.subcore_barrier()
            pltpu.sync_copy(acc.at[rslice], out_h.at[core, ch, sub])
            plsc.subcore_barrier()

    part = k(*tbl_chunks, col3, row3, w2, zeros)
    return part.reshape(2, nch, n, 128)


def kernel(x, edge_index, edge_weight, params):
    p = params
    n = x.shape[0]

    col3 = edge_index[1].reshape(NTILES, NB, 1, EB)
    row3 = edge_index[0].reshape(NTILES, NB, 1, EB)
    w2 = edge_weight.reshape(NTILES, NB, EB)

    h1 = _lin_bn(x, p["enc_1_W"], p["enc_1_b"], p["BN1_g"], p["BN1_b"], True)
    h2 = _lin_bn(h1, p["enc_2_W"], p["enc_2_b"], p["BN2_g"], p["BN2_b"], True)
    h3 = _lin_bn(h2, p["enc_3_W"], p["enc_3_b"], p["BN3_g"], p["BN3_b"], True)
    z1 = _lin_bn(h3, p["z1_W"], p["z1_b"], p["BN4_g"], p["BN4_b"], False)
    z2 = _lin_bn(z1, p["z2_W"], p["z2_b"], p["BN5_g"], p["BN5_b"], False)
    z3 = _lin_bn(z2, p["z3_W"], p["z3_b"], p["BN6_g"], p["BN6_b"], False)
    d1 = _lin_bn(z3, p["dec_1_W"], p["dec_1_b"], p["BN7_g"], p["BN7_b"], True)
    d2 = _lin_bn(d1, p["dec_2_W"], p["dec_2_b"], p["BN8_g"], p["BN8_b"], True)
    d3 = _lin_bn(d2, p["dec_3_W"], p["dec_3_b"], p["BN9_g"], p["BN9_b"], True)
    x_bar, mean, disp, pi = _heads(d3, p)

    tras = [h1, h2, h3, z1, z2, z3]
    zeros = jnp.zeros((ROWS_PER_TILE, 128), jnp.float32)

    part = _sc_spmm([x], col3, row3, w2, zeros, 128)
    hmix = _postmm_mix(part, p["gnn_1_W"], tras[0])
    for i in range(2, 8):
        gw = p["gnn_%d_W" % i]
        fo = gw.shape[1]
        bw = 128 if fo % 128 == 0 else fo
        tm = _matmul_chunked(hmix, gw, bw)
        if bw < 128:
            chunks = [jnp.pad(tm[0], ((0, 0), (0, 128 - bw)))]
        else:
            chunks = [tm[c] for c in range(fo // 128)]
        part = _sc_spmm(chunks, col3, row3, w2, zeros, bw)
        part = part[:, :, :, :bw]
        if i < 7:
            hmix = _mix(part, tras[i - 1])
    predict = _softmax_of_part(part)

    q = _q_kernel(z3, p["cluster"].T)

    return (x_bar, q, predict, z3, mean, disp, pi)


# speedup vs baseline: 5.8115x; 1.0013x over previous
import dataclasses
import functools

import jax
import jax.numpy as jnp
from jax import lax
from jax.experimental import pallas as pl
from jax.experimental.pallas import tpu as pltpu
from jax.experimental.pallas import tpu_sc as plsc

SIGMA = 0.5
V = 1.0
N_NODES = 10000
N_EDGES = 160000
NTILES = 32
EDGES_PER_TILE = N_EDGES // NTILES
EB = 250
NB = EDGES_PER_TILE // EB
ROWS_PER_TILE = N_NODES // 16


def _linbn_body(x_ref, w_ref, b_ref, g_ref, bb_ref, o_ref, *, relu):
    y = jnp.dot(x_ref[...], w_ref[...], preferred_element_type=jnp.float32)
    y = y + b_ref[...]
    mu = jnp.mean(y, axis=0, keepdims=True)
    var = jnp.mean((y - mu) ** 2, axis=0, keepdims=True)
    y = g_ref[...] * (y - mu) / jnp.sqrt(var + 1e-5) + bb_ref[...]
    if relu:
        y = jnp.maximum(y, 0.0)
    o_ref[...] = y


def _lin_bn(x, w, b, g, bb, relu):
    n, k = x.shape
    fo = w.shape[1]
    bw = 128 if fo % 128 == 0 else fo
    grid = (fo // bw,)
    return pl.pallas_call(
        functools.partial(_linbn_body, relu=relu),
        grid=grid,
        in_specs=[
            pl.BlockSpec((n, k), lambda c: (0, 0)),
            pl.BlockSpec((k, bw), lambda c: (0, c)),
            pl.BlockSpec((bw,), lambda c: (c,)),
            pl.BlockSpec((bw,), lambda c: (c,)),
            pl.BlockSpec((bw,), lambda c: (c,)),
        ],
        out_specs=pl.BlockSpec((n, bw), lambda c: (0, c)),
        out_shape=jax.ShapeDtypeStruct((n, fo), jnp.float32),
    )(x, w, b, g, bb)


def _mm_chunk_body(h_ref, w_ref, o_ref):
    o_ref[0] = jnp.dot(h_ref[...], w_ref[...], preferred_element_type=jnp.float32)


def _matmul_chunked(h, w, fc):
    n, k = h.shape
    fo = w.shape[1]
    nch = fo // fc
    return pl.pallas_call(
        _mm_chunk_body,
        grid=(nch,),
        in_specs=[
            pl.BlockSpec((n, k), lambda c: (0, 0)),
            pl.BlockSpec((k, fc), lambda c: (0, c)),
        ],
        out_specs=pl.BlockSpec((1, n, fc), lambda c: (c, 0, 0)),
        out_shape=jax.ShapeDtypeStruct((nch, n, fc), jnp.float32),
    )(h, w)


def _postmm_body(p_ref, w_ref, t_ref, o_ref, *, nchp):
    h = p_ref[0, 0] + p_ref[1, 0]
    if nchp > 1:
        h = jnp.concatenate(
            [h] + [p_ref[0, c] + p_ref[1, c] for c in range(1, nchp)], axis=1)
    y = jnp.maximum(jnp.dot(h, w_ref[...], preferred_element_type=jnp.float32),
                    0.0)
    o_ref[...] = (1.0 - SIGMA) * y + SIGMA * t_ref[...]


def _postmm_mix(part, w, tra):
    _, nchp, n, fcp = part.shape
    fo = w.shape[1]
    bw = 128 if fo % 128 == 0 else fo
    return pl.pallas_call(
        functools.partial(_postmm_body, nchp=nchp),
        grid=(fo // bw,),
        in_specs=[
            pl.BlockSpec((2, nchp, n, fcp), lambda c: (0, 0, 0, 0)),
            pl.BlockSpec((nchp * fcp, bw), lambda c: (0, c)),
            pl.BlockSpec((n, bw), lambda c: (0, c)),
        ],
        out_specs=pl.BlockSpec((n, bw), lambda c: (0, c)),
        out_shape=jax.ShapeDtypeStruct((n, fo), jnp.float32),
    )(part, w, tra)


def _mix_body(p0_ref, p1_ref, t_ref, o_ref, *, kk):
    h = p0_ref[0, 0] + p1_ref[0, 0]
    if kk > 1:
        h = jnp.concatenate(
            [h] + [p0_ref[0, c] + p1_ref[0, c] for c in range(1, kk)], axis=1)
    h = jnp.maximum(h, 0.0)
    o_ref[...] = (1.0 - SIGMA) * h + SIGMA * t_ref[...]


def _mix(part, tra):
    _, nch, n, fc = part.shape
    fo = nch * fc
    bw = 128 if fo % 128 == 0 else fo
    kk = bw // fc
    return pl.pallas_call(
        functools.partial(_mix_body, kk=kk),
        grid=(fo // bw,),
        in_specs=[
            pl.BlockSpec((1, kk, n, fc), lambda c: (0, c, 0, 0)),
            pl.BlockSpec((1, kk, n, fc), lambda c: (1, c, 0, 0)),
            pl.BlockSpec((n, bw), lambda c: (0, c)),
        ],
        out_specs=pl.BlockSpec((n, bw), lambda c: (0, c)),
        out_shape=jax.ShapeDtypeStruct((n, fo), jnp.float32),
    )(part[:, :, :, :], part, tra)


def _softmax_body(p0_ref, p1_ref, o_ref):
    h = p0_ref[...] + p1_ref[...]
    m = jnp.max(h, axis=1, keepdims=True)
    e = jnp.exp(h - m)
    o_ref[...] = e / jnp.sum(e, axis=1, keepdims=True)


def _softmax_of_part(part):
    _, nch, n, fc = part.shape
    return pl.pallas_call(
        _softmax_body,
        out_shape=jax.ShapeDtypeStruct((n, fc), jnp.float32),
    )(part[0, 0], part[1, 0])


def _heads_body(d_ref, wx_ref, bx_ref, wm_ref, bm_ref, wd_ref, bd_ref,
                wp_ref, bp_ref, xb_ref, mean_ref, disp_ref, pi_ref):
    d = d_ref[...]
    xb_ref[...] = jnp.dot(d, wx_ref[...], preferred_element_type=jnp.float32) + bx_ref[...]
    ym = jnp.dot(d, wm_ref[...], preferred_element_type=jnp.float32) + bm_ref[...]
    mean_ref[...] = jnp.clip(jnp.exp(ym), 1e-5, 1e6)
    yd = jnp.dot(d, wd_ref[...], preferred_element_type=jnp.float32) + bd_ref[...]
    disp_ref[...] = jnp.clip(jnp.log1p(jnp.exp(-jnp.abs(yd))) + jnp.maximum(yd, 0.0),
                             1e-4, 1e4)
    yp = jnp.dot(d, wp_ref[...], preferred_element_type=jnp.float32) + bp_ref[...]
    pi_ref[...] = 1.0 / (1.0 + jnp.exp(-yp))


def _heads(d3, p):
    n = d3.shape[0]
    fo = p["x_bar_W"].shape[1]
    outs = [jax.ShapeDtypeStruct((n, fo), jnp.float32)] * 4
    return pl.pallas_call(
        _heads_body,
        out_shape=outs,
    )(d3, p["x_bar_W"], p["x_bar_b"], p["dmean_W"], p["dmean_b"],
      p["ddisp_W"], p["ddisp_b"], p["dpi_W"], p["dpi_b"])


def _q_body(z_ref, ct_ref, o_ref):
    z = z_ref[...]
    ct = ct_ref[...]
    zz = jnp.sum(z * z, axis=1, keepdims=True)
    cc = jnp.sum(ct * ct, axis=0, keepdims=True)
    cross = jnp.dot(z, ct, preferred_element_type=jnp.float32)
    q = 1.0 / (1.0 + (zz + cc - 2.0 * cross) / V)
    o_ref[...] = q / jnp.sum(q, axis=1, keepdims=True)


def _q_kernel(z3, cluster_t):
    n = z3.shape[0]
    nc = cluster_t.shape[1]
    return pl.pallas_call(
        _q_body,
        out_shape=jax.ShapeDtypeStruct((n, nc), jnp.float32),
    )(z3, cluster_t)


def _sc_spmm(tbl_chunks, col3, row3, w2, zeros, width):
    nch = len(tbl_chunks)
    n = 16 * ROWS_PER_TILE
    ngroups = width // 16
    mesh = plsc.VectorSubcoreMesh(core_axis_name="c", subcore_axis_name="s")
    cp = pltpu.CompilerParams()
    fields = pltpu.CompilerParams.__dataclass_fields__
    if "needs_layout_passes" in fields:
        cp = dataclasses.replace(cp, needs_layout_passes=False)

    @functools.partial(
        pl.kernel,
        compiler_params=cp,
        out_type=jax.ShapeDtypeStruct((2, nch, 16, ROWS_PER_TILE, 128),
                                      jnp.float32),
        mesh=mesh,
        scratch_types=[
            pltpu.VMEM((NB, 1, EB), jnp.int32),
            pltpu.VMEM((NB, 1, EB), jnp.int32),
            pltpu.VMEM((NB, EB), jnp.float32),
            pltpu.VMEM((EB, 128), jnp.float32),
            pltpu.VMEM_SHARED((n, 128), jnp.float32),
            pltpu.SemaphoreType.DMA,
        ],
    )
    def k(*refs):
        tbls = refs[:nch]
        col_h, row_h, w_h, z_h, out_h = refs[nch:nch + 5]
        col_v, row_v, w_v, rv0, acc, sg0 = refs[nch + 5:]
        core = lax.axis_index("c")
        sub = lax.axis_index("s")
        wid = sub * 2 + core
        rslice = pl.ds(sub * ROWS_PER_TILE, ROWS_PER_TILE)
        pltpu.sync_copy(col_h.at[wid], col_v)
        pltpu.sync_copy(row_h.at[wid], row_v)
        pltpu.sync_copy(w_h.at[wid], w_v)

        for ch in range(nch):
            tb = tbls[ch]

            def scale(b, rv):
                @functools.partial(plsc.parallel_loop, 0, EB, unroll=10)
                def _edge(e):
                    wv = plsc.load_gather(
                        w_v, [jnp.full((16,), b, jnp.int32),
                              jnp.full((16,), e, jnp.int32)])
                    for f in range(ngroups):
                        sl = pl.ds(f * 16, 16)
                        rv[e, sl] = rv[e, sl] * wv

            pltpu.sync_copy(z_h, acc.at[rslice])
            plsc.subcore_barrier()

            @pl.loop(0, NB)
            def _batch(b):
                pltpu.async_copy(tb.at[col_v.at[b].at[0]], rv0, sg0).wait()
                scale(b, rv0)
                pltpu.sync_copy(rv0, acc.at[row_v.at[b].at[0]], add=True)

            plsc.subcore_barrier()
            pltpu.sync_copy(acc.at[rslice], out_h.at[core, ch, sub])
            plsc.subcore_barrier()

    part = k(*tbl_chunks, col3, row3, w2, zeros)
    return part.reshape(2, nch, n, 128)


def kernel(x, edge_index, edge_weight, params):
    p = params
    n = x.shape[0]

    col3 = edge_index[1].reshape(NTILES, NB, 1, EB)
    row3 = edge_index[0].reshape(NTILES, NB, 1, EB)
    w2 = edge_weight.reshape(NTILES, NB, EB)

    h1 = _lin_bn(x, p["enc_1_W"], p["enc_1_b"], p["BN1_g"], p["BN1_b"], True)
    h2 = _lin_bn(h1, p["enc_2_W"], p["enc_2_b"], p["BN2_g"], p["BN2_b"], True)
    h3 = _lin_bn(h2, p["enc_3_W"], p["enc_3_b"], p["BN3_g"], p["BN3_b"], True)
    z1 = _lin_bn(h3, p["z1_W"], p["z1_b"], p["BN4_g"], p["BN4_b"], False)
    z2 = _lin_bn(z1, p["z2_W"], p["z2_b"], p["BN5_g"], p["BN5_b"], False)
    z3 = _lin_bn(z2, p["z3_W"], p["z3_b"], p["BN6_g"], p["BN6_b"], False)
    d1 = _lin_bn(z3, p["dec_1_W"], p["dec_1_b"], p["BN7_g"], p["BN7_b"], True)
    d2 = _lin_bn(d1, p["dec_2_W"], p["dec_2_b"], p["BN8_g"], p["BN8_b"], True)
    d3 = _lin_bn(d2, p["dec_3_W"], p["dec_3_b"], p["BN9_g"], p["BN9_b"], True)
    x_bar, mean, disp, pi = _heads(d3, p)

    tras = [h1, h2, h3, z1, z2, z3]
    zeros = jnp.zeros((ROWS_PER_TILE, 128), jnp.float32)

    part = _sc_spmm([x], col3, row3, w2, zeros, 128)
    hmix = _postmm_mix(part, p["gnn_1_W"], tras[0])
    for i in range(2, 8):
        gw = p["gnn_%d_W" % i]
        fo = gw.shape[1]
        bw = 128 if fo % 128 == 0 else fo
        tm = _matmul_chunked(hmix, gw, bw)
        if bw < 128:
            chunks = [jnp.pad(tm[0], ((0, 0), (0, 128 - bw)))]
        else:
            chunks = [tm[c] for c in range(fo // 128)]
        part = _sc_spmm(chunks, col3, row3, w2, zeros, bw)
        part = part[:, :, :, :bw]
        if i < 7:
            hmix = _mix(part, tras[i - 1])
    predict = _softmax_of_part(part)

    q = _q_kernel(z3, p["cluster"].T)

    return (x_bar, q, predict, z3, mean, disp, pi)
